# baseline (speedup 1.0000x reference)
import functools

import jax
import jax.numpy as jnp
from jax import lax
from jax.experimental import pallas as pl
from jax.experimental.pallas import tpu as pltpu
from jax.experimental.pallas import tpu_sc as plsc

_NC = 2
_NS = 16
_NW = _NC * _NS
_LANES = 16


def _relayout_table(table):
    V, D = table.shape
    CW = 128
    NBLK = (V + CW - 1) // CW

    def body(t_ref, out_ref):
        x = t_ref[...].reshape(D, D, CW // D)
        out_ref[...] = jnp.transpose(x, (1, 2, 0)).reshape(D, CW)

    wide = pl.pallas_call(
        body,
        grid=(NBLK,),
        in_specs=[pl.BlockSpec((D, CW), lambda m: (0, m))],
        out_specs=pl.BlockSpec((D, CW), lambda m: (m, 0)),
        out_shape=jax.ShapeDtypeStruct((NBLK * D, CW), jnp.float32),
    )(table.T)
    return wide.reshape(NBLK * D * CW)[: V * D].reshape(V, D)


def kernel(feats, weights, table):
    B, H = feats.shape
    V, D = table.shape
    CB = 2
    RB = B // _NW
    NCHUNK = RB // CB
    NBUF = 4
    RPC = CB * H

    HP = 64
    feats2 = feats.reshape(B // CB, RPC).astype(jnp.int32)
    weights2 = jnp.pad(weights, ((0, 0), (0, HP - H))).reshape(B // CB, CB * HP)
    table_lin = _relayout_table(table)

    mesh = plsc.VectorSubcoreMesh(core_axis_name="c", subcore_axis_name="s")

    @functools.partial(
        pl.kernel,
        out_type=jax.ShapeDtypeStruct((B, D), jnp.float32),
        mesh=mesh,
        scratch_types=[
            pltpu.VMEM((NCHUNK, RPC), jnp.int32),
            pltpu.VMEM((NCHUNK, CB * HP), jnp.float32),
            pltpu.VMEM((NBUF, RPC, D), jnp.float32),
            pltpu.VMEM((RB, D), jnp.float32),
            pltpu.SemaphoreType.DMA,
        ],
        compiler_params=pltpu.CompilerParams(use_tc_tiling_on_sc=False),
    )
    def run(feats_hbm, w_hbm, table_hbm, out_hbm, idx_v, w_v, rows_v, out_v, sem):
        wid = lax.axis_index("s") * _NC + lax.axis_index("c")
        chunk0 = wid * NCHUNK

        pltpu.sync_copy(feats_hbm.at[pl.ds(chunk0, NCHUNK)], idx_v)
        pltpu.sync_copy(w_hbm.at[pl.ds(chunk0, NCHUNK)], w_v)

        def fire(g, b):
            pltpu.async_copy(table_hbm.at[idx_v.at[g]], rows_v.at[b], sem)

        def wait(g, b):
            pltpu.make_async_copy(
                table_hbm.at[idx_v.at[g]], rows_v.at[b], sem).wait()

        def compute(g, b):
            for cb in range(CB):
                wvecs = [w_v[g, pl.ds(cb * HP + o, _LANES)] for o in (0, 16, 32, 40)]
                acc0 = jnp.zeros((_LANES,), jnp.float32)
                acc1 = jnp.zeros((_LANES,), jnp.float32)
                for l in range(H):
                    r = cb * H + l
                    w = wvecs[l // 16][l % 16] if l < 48 else wvecs[3][l - 40]
                    acc0 = acc0 + w * rows_v[b, r, pl.ds(0, _LANES)]
                    acc1 = acc1 + w * rows_v[b, r, pl.ds(_LANES, _LANES)]
                out_v[g * CB + cb, pl.ds(0, _LANES)] = acc0
                out_v[g * CB + cb, pl.ds(_LANES, _LANES)] = acc1

        for b in range(NBUF):
            fire(b, b)

        @pl.loop(0, NCHUNK - NBUF, step=NBUF)
        def _(g0):
            for b in range(NBUF):
                g = g0 + b
                wait(g, b)
                compute(g, b)
                fire(g + NBUF, b)

        for b in range(NBUF):
            g = NCHUNK - NBUF + b
            wait(g, b)
            compute(g, b)

        pltpu.sync_copy(out_v, out_hbm.at[pl.ds(wid * RB, RB)])

    return run(feats2, weights2, table_lin)


# speedup vs baseline: 12.2061x; 12.2061x over previous
import functools

import jax
import jax.numpy as jnp
from jax import lax
from jax.experimental import pallas as pl
from jax.experimental.pallas import tpu as pltpu
from jax.experimental.pallas import tpu_sc as plsc

_NC = 2
_NS = 16
_NW = _NC * _NS
_LANES = 16


def _relayout_table(table):
    V, D = table.shape
    CW = 128
    QB = 16
    BW_ = QB * CW
    NBLK = (V + BW_ - 1) // BW_

    def body(t_ref, out_ref):
        x = t_ref[...]
        for h in range(QB // 4):
            z = jnp.concatenate(
                [x[:, (4 * h + i) * CW:(4 * h + i + 1) * CW] for i in range(4)],
                axis=0)
            out_ref[pl.ds(h * CW, CW), :] = jnp.transpose(z)

    wide = pl.pallas_call(
        body,
        grid=(NBLK,),
        in_specs=[pl.BlockSpec((D, BW_), lambda m: (0, m))],
        out_specs=pl.BlockSpec((QB // 4 * CW, CW), lambda m: (m, 0)),
        out_shape=jax.ShapeDtypeStruct((NBLK * QB // 4 * CW, CW), jnp.float32),
    )(table.T)
    return wide.reshape(NBLK * BW_, D)


def kernel(feats, weights, table):
    B, H = feats.shape
    V, D = table.shape
    CB = 2
    RB = B // _NW
    NCHUNK = RB // CB
    NBUF = 4
    RPC = CB * H

    HP = 64
    fj = feats.astype(jnp.int32)
    fr = ((fj >> 9) << 9) + ((fj & 127) << 2) + ((fj >> 7) & 3)
    feats2 = fr.reshape(B // CB, RPC)
    weights2 = jnp.pad(weights, ((0, 0), (0, HP - H))).reshape(B // CB, CB * HP)
    table_lin = _relayout_table(table)

    mesh = plsc.VectorSubcoreMesh(core_axis_name="c", subcore_axis_name="s")

    @functools.partial(
        pl.kernel,
        out_type=jax.ShapeDtypeStruct((B, D), jnp.float32),
        mesh=mesh,
        scratch_types=[
            pltpu.VMEM((NCHUNK, RPC), jnp.int32),
            pltpu.VMEM((NCHUNK, CB * HP), jnp.float32),
            pltpu.VMEM((NBUF, RPC, D), jnp.float32),
            pltpu.VMEM((RB, D), jnp.float32),
            pltpu.SemaphoreType.DMA,
        ],
        compiler_params=pltpu.CompilerParams(use_tc_tiling_on_sc=False),
    )
    def run(feats_hbm, w_hbm, table_hbm, out_hbm, idx_v, w_v, rows_v, out_v, sem):
        wid = lax.axis_index("s") * _NC + lax.axis_index("c")
        chunk0 = wid * NCHUNK

        pltpu.sync_copy(feats_hbm.at[pl.ds(chunk0, NCHUNK)], idx_v)
        pltpu.sync_copy(w_hbm.at[pl.ds(chunk0, NCHUNK)], w_v)

        def fire(g, b):
            pltpu.async_copy(table_hbm.at[idx_v.at[g]], rows_v.at[b], sem)

        def wait(g, b):
            pltpu.make_async_copy(
                table_hbm.at[idx_v.at[g]], rows_v.at[b], sem).wait()

        def compute(g, b):
            for cb in range(CB):
                wvecs = [w_v[g, pl.ds(cb * HP + o, _LANES)] for o in (0, 16, 32, 40)]
                acc0 = jnp.zeros((_LANES,), jnp.float32)
                acc1 = jnp.zeros((_LANES,), jnp.float32)
                for l in range(H):
                    r = cb * H + l
                    w = wvecs[l // 16][l % 16] if l < 48 else wvecs[3][l - 40]
                    acc0 = acc0 + w * rows_v[b, r, pl.ds(0, _LANES)]
                    acc1 = acc1 + w * rows_v[b, r, pl.ds(_LANES, _LANES)]
                out_v[g * CB + cb, pl.ds(0, _LANES)] = acc0
                out_v[g * CB + cb, pl.ds(_LANES, _LANES)] = acc1

        for b in range(NBUF):
            fire(b, b)

        @pl.loop(0, NCHUNK - NBUF, step=NBUF)
        def _(g0):
            for b in range(NBUF):
                g = g0 + b
                wait(g, b)
                compute(g, b)
                fire(g + NBUF, b)

        for b in range(NBUF):
            g = NCHUNK - NBUF + b
            wait(g, b)
            compute(g, b)

        pltpu.sync_copy(out_v, out_hbm.at[pl.ds(wid * RB, RB)])

    return run(feats2, weights2, table_lin)


# speedup vs baseline: 16.8962x; 1.3842x over previous
import functools

import jax
import jax.numpy as jnp
from jax import lax
from jax.experimental import pallas as pl
from jax.experimental.pallas import tpu as pltpu
from jax.experimental.pallas import tpu_sc as plsc

_NC = 2
_NS = 16
_NW = _NC * _NS
_LANES = 16


def _relayout_table(table):
    V, D = table.shape
    CW = 128
    QB = 32
    BW_ = QB * CW
    NBLK = (V + BW_ - 1) // BW_

    def body(t_ref, out_ref):
        x = t_ref[...]
        for h in range(QB // 4):
            z = jnp.concatenate(
                [x[:, (4 * h + i) * CW:(4 * h + i + 1) * CW] for i in range(4)],
                axis=0)
            out_ref[pl.ds(h * CW, CW), :] = jnp.transpose(z)

    wide = pl.pallas_call(
        body,
        grid=(NBLK,),
        in_specs=[pl.BlockSpec((D, BW_), lambda m: (0, m))],
        out_specs=pl.BlockSpec((QB // 4 * CW, CW), lambda m: (m, 0)),
        out_shape=jax.ShapeDtypeStruct((NBLK * QB // 4 * CW, CW), jnp.float32),
    )(table.T)
    return wide.reshape(NBLK * BW_, D)


def kernel(feats, weights, table):
    B, H = feats.shape
    V, D = table.shape
    CB = 2
    RB = B // _NW
    NCHUNK = RB // CB
    NBUF = 4
    RPC = CB * H

    HP = 64
    fj = feats.astype(jnp.int32)
    fr = ((fj >> 9) << 9) + ((fj & 127) << 2) + ((fj >> 7) & 3)
    feats2 = fr.reshape(B // CB, RPC)
    weights2 = jnp.pad(weights, ((0, 0), (0, HP - H))).reshape(B // CB, CB * HP)
    table_lin = _relayout_table(table)

    mesh = plsc.VectorSubcoreMesh(core_axis_name="c", subcore_axis_name="s")

    @functools.partial(
        pl.kernel,
        out_type=jax.ShapeDtypeStruct((B, D), jnp.float32),
        mesh=mesh,
        scratch_types=[
            pltpu.VMEM((NCHUNK, RPC), jnp.int32),
            pltpu.VMEM((NCHUNK, CB * HP), jnp.float32),
            pltpu.VMEM((NBUF, RPC, D), jnp.float32),
            pltpu.VMEM((RB, D), jnp.float32),
            pltpu.SemaphoreType.DMA,
        ],
        compiler_params=pltpu.CompilerParams(use_tc_tiling_on_sc=False),
    )
    def run(feats_hbm, w_hbm, table_hbm, out_hbm, idx_v, w_v, rows_v, out_v, sem):
        wid = lax.axis_index("s") * _NC + lax.axis_index("c")
        chunk0 = wid * NCHUNK

        pltpu.sync_copy(feats_hbm.at[pl.ds(chunk0, NCHUNK)], idx_v)
        pltpu.sync_copy(w_hbm.at[pl.ds(chunk0, NCHUNK)], w_v)

        def fire(g, b):
            pltpu.async_copy(table_hbm.at[idx_v.at[g]], rows_v.at[b], sem)

        def wait(g, b):
            pltpu.make_async_copy(
                table_hbm.at[idx_v.at[g]], rows_v.at[b], sem).wait()

        def compute(g, b):
            for cb in range(CB):
                wvecs = [w_v[g, pl.ds(cb * HP + o, _LANES)] for o in (0, 16, 32, 40)]
                acc0 = jnp.zeros((_LANES,), jnp.float32)
                acc1 = jnp.zeros((_LANES,), jnp.float32)
                for l in range(H):
                    r = cb * H + l
                    w = wvecs[l // 16][l % 16] if l < 48 else wvecs[3][l - 40]
                    acc0 = acc0 + w * rows_v[b, r, pl.ds(0, _LANES)]
                    acc1 = acc1 + w * rows_v[b, r, pl.ds(_LANES, _LANES)]
                out_v[g * CB + cb, pl.ds(0, _LANES)] = acc0
                out_v[g * CB + cb, pl.ds(_LANES, _LANES)] = acc1

        for b in range(NBUF):
            fire(b, b)

        @pl.loop(0, NCHUNK - NBUF, step=NBUF)
        def _(g0):
            for b in range(NBUF):
                g = g0 + b
                wait(g, b)
                compute(g, b)
                fire(g + NBUF, b)

        for b in range(NBUF):
            g = NCHUNK - NBUF + b
            wait(g, b)
            compute(g, b)

        pltpu.sync_copy(out_v, out_hbm.at[pl.ds(wid * RB, RB)])

    return run(feats2, weights2, table_lin)


# speedup vs baseline: 20.4655x; 1.2112x over previous
import functools

import jax
import jax.numpy as jnp
from jax import lax
from jax.experimental import pallas as pl
from jax.experimental.pallas import tpu as pltpu
from jax.experimental.pallas import tpu_sc as plsc

_NC = 2
_NS = 16
_NW = _NC * _NS
_LANES = 16


def _relayout_table(table):
    V, D = table.shape
    CW = 128
    QB = 64
    BW_ = QB * CW
    NBLK = (V + BW_ - 1) // BW_

    def body(t_ref, out_ref):
        x = t_ref[...]
        for h in range(QB // 4):
            z = jnp.concatenate(
                [x[:, (4 * h + i) * CW:(4 * h + i + 1) * CW] for i in range(4)],
                axis=0)
            out_ref[pl.ds(h * CW, CW), :] = jnp.transpose(z)

    wide = pl.pallas_call(
        body,
        grid=(NBLK,),
        in_specs=[pl.BlockSpec((D, BW_), lambda m: (0, m))],
        out_specs=pl.BlockSpec((QB // 4 * CW, CW), lambda m: (m, 0)),
        out_shape=jax.ShapeDtypeStruct((NBLK * QB // 4 * CW, CW), jnp.float32),
    )(table.T)
    return wide.reshape(NBLK * BW_, D)


def kernel(feats, weights, table):
    B, H = feats.shape
    V, D = table.shape
    CB = 2
    RB = B // _NW
    NCHUNK = RB // CB
    NBUF = 4
    RPC = CB * H

    HP = 64
    fj = feats.astype(jnp.int32)
    fr = ((fj >> 9) << 9) + ((fj & 127) << 2) + ((fj >> 7) & 3)
    feats2 = fr.reshape(B // CB, RPC)
    weights2 = jnp.pad(weights, ((0, 0), (0, HP - H))).reshape(B // CB, CB * HP)
    table_lin = _relayout_table(table)

    mesh = plsc.VectorSubcoreMesh(core_axis_name="c", subcore_axis_name="s")

    @functools.partial(
        pl.kernel,
        out_type=jax.ShapeDtypeStruct((B, D), jnp.float32),
        mesh=mesh,
        scratch_types=[
            pltpu.VMEM((NCHUNK, RPC), jnp.int32),
            pltpu.VMEM((NCHUNK, CB * HP), jnp.float32),
            pltpu.VMEM((NBUF, RPC, D), jnp.float32),
            pltpu.VMEM((RB, D), jnp.float32),
            pltpu.SemaphoreType.DMA,
        ],
        compiler_params=pltpu.CompilerParams(use_tc_tiling_on_sc=False),
    )
    def run(feats_hbm, w_hbm, table_hbm, out_hbm, idx_v, w_v, rows_v, out_v, sem):
        wid = lax.axis_index("s") * _NC + lax.axis_index("c")
        chunk0 = wid * NCHUNK

        pltpu.sync_copy(feats_hbm.at[pl.ds(chunk0, NCHUNK)], idx_v)
        pltpu.sync_copy(w_hbm.at[pl.ds(chunk0, NCHUNK)], w_v)

        def fire(g, b):
            pltpu.async_copy(table_hbm.at[idx_v.at[g]], rows_v.at[b], sem)

        def wait(g, b):
            pltpu.make_async_copy(
                table_hbm.at[idx_v.at[g]], rows_v.at[b], sem).wait()

        def compute(g, b):
            for cb in range(CB):
                wvecs = [w_v[g, pl.ds(cb * HP + o, _LANES)] for o in (0, 16, 32, 40)]
                acc0 = jnp.zeros((_LANES,), jnp.float32)
                acc1 = jnp.zeros((_LANES,), jnp.float32)
                for l in range(H):
                    r = cb * H + l
                    w = wvecs[l // 16][l % 16] if l < 48 else wvecs[3][l - 40]
                    acc0 = acc0 + w * rows_v[b, r, pl.ds(0, _LANES)]
                    acc1 = acc1 + w * rows_v[b, r, pl.ds(_LANES, _LANES)]
                out_v[g * CB + cb, pl.ds(0, _LANES)] = acc0
                out_v[g * CB + cb, pl.ds(_LANES, _LANES)] = acc1

        for b in range(NBUF):
            fire(b, b)

        @pl.loop(0, NCHUNK - NBUF, step=NBUF)
        def _(g0):
            for b in range(NBUF):
                g = g0 + b
                wait(g, b)
                compute(g, b)
                fire(g + NBUF, b)

        for b in range(NBUF):
            g = NCHUNK - NBUF + b
            wait(g, b)
            compute(g, b)

        pltpu.sync_copy(out_v, out_hbm.at[pl.ds(wid * RB, RB)])

    return run(feats2, weights2, table_lin)


# speedup vs baseline: 23.6798x; 1.1571x over previous
import functools

import jax
import jax.numpy as jnp
from jax import lax
from jax.experimental import pallas as pl
from jax.experimental.pallas import tpu as pltpu
from jax.experimental.pallas import tpu_sc as plsc

_NC = 2
_NS = 16
_NW = _NC * _NS
_LANES = 16


def _relayout_table(table):
    V, D = table.shape
    CW = 128
    QB = 128
    BW_ = QB * CW
    NBLK = (V + BW_ - 1) // BW_

    def body(t_ref, out_ref):
        x = t_ref[...]
        for h in range(QB // 4):
            z = jnp.concatenate(
                [x[:, (4 * h + i) * CW:(4 * h + i + 1) * CW] for i in range(4)],
                axis=0)
            out_ref[pl.ds(h * CW, CW), :] = jnp.transpose(z)

    wide = pl.pallas_call(
        body,
        grid=(NBLK,),
        in_specs=[pl.BlockSpec((D, BW_), lambda m: (0, m))],
        out_specs=pl.BlockSpec((QB // 4 * CW, CW), lambda m: (m, 0)),
        out_shape=jax.ShapeDtypeStruct((NBLK * QB // 4 * CW, CW), jnp.float32),
    )(table.T)
    return wide.reshape(NBLK * BW_, D)


def kernel(feats, weights, table):
    B, H = feats.shape
    V, D = table.shape
    CB = 2
    RB = B // _NW
    NCHUNK = RB // CB
    NBUF = 4
    RPC = CB * H

    HP = 64
    fj = feats.astype(jnp.int32)
    fr = ((fj >> 9) << 9) + ((fj & 127) << 2) + ((fj >> 7) & 3)
    feats2 = fr.reshape(B // CB, RPC)
    weights2 = jnp.pad(weights, ((0, 0), (0, HP - H))).reshape(B // CB, CB * HP)
    table_lin = _relayout_table(table)

    mesh = plsc.VectorSubcoreMesh(core_axis_name="c", subcore_axis_name="s")

    @functools.partial(
        pl.kernel,
        out_type=jax.ShapeDtypeStruct((B, D), jnp.float32),
        mesh=mesh,
        scratch_types=[
            pltpu.VMEM((NCHUNK, RPC), jnp.int32),
            pltpu.VMEM((NCHUNK, CB * HP), jnp.float32),
            pltpu.VMEM((NBUF, RPC, D), jnp.float32),
            pltpu.VMEM((RB, D), jnp.float32),
            pltpu.SemaphoreType.DMA,
        ],
        compiler_params=pltpu.CompilerParams(use_tc_tiling_on_sc=False),
    )
    def run(feats_hbm, w_hbm, table_hbm, out_hbm, idx_v, w_v, rows_v, out_v, sem):
        wid = lax.axis_index("s") * _NC + lax.axis_index("c")
        chunk0 = wid * NCHUNK

        pltpu.sync_copy(feats_hbm.at[pl.ds(chunk0, NCHUNK)], idx_v)
        pltpu.sync_copy(w_hbm.at[pl.ds(chunk0, NCHUNK)], w_v)

        def fire(g, b):
            pltpu.async_copy(table_hbm.at[idx_v.at[g]], rows_v.at[b], sem)

        def wait(g, b):
            pltpu.make_async_copy(
                table_hbm.at[idx_v.at[g]], rows_v.at[b], sem).wait()

        def compute(g, b):
            for cb in range(CB):
                wvecs = [w_v[g, pl.ds(cb * HP + o, _LANES)] for o in (0, 16, 32, 40)]
                acc0 = jnp.zeros((_LANES,), jnp.float32)
                acc1 = jnp.zeros((_LANES,), jnp.float32)
                for l in range(H):
                    r = cb * H + l
                    w = wvecs[l // 16][l % 16] if l < 48 else wvecs[3][l - 40]
                    acc0 = acc0 + w * rows_v[b, r, pl.ds(0, _LANES)]
                    acc1 = acc1 + w * rows_v[b, r, pl.ds(_LANES, _LANES)]
                out_v[g * CB + cb, pl.ds(0, _LANES)] = acc0
                out_v[g * CB + cb, pl.ds(_LANES, _LANES)] = acc1

        for b in range(NBUF):
            fire(b, b)

        @pl.loop(0, NCHUNK - NBUF, step=NBUF)
        def _(g0):
            for b in range(NBUF):
                g = g0 + b
                wait(g, b)
                compute(g, b)
                fire(g + NBUF, b)

        for b in range(NBUF):
            g = NCHUNK - NBUF + b
            wait(g, b)
            compute(g, b)

        pltpu.sync_copy(out_v, out_hbm.at[pl.ds(wid * RB, RB)])

    return run(feats2, weights2, table_lin)


# speedup vs baseline: 24.7513x; 1.0452x over previous
import functools

import jax
import jax.numpy as jnp
from jax import lax
from jax.experimental import pallas as pl
from jax.experimental.pallas import tpu as pltpu
from jax.experimental.pallas import tpu_sc as plsc

_NC = 2
_NS = 16
_NW = _NC * _NS
_LANES = 16


def _relayout_table(table):
    V, D = table.shape
    CW = 128
    QB = 256
    BW_ = QB * CW
    NBLK = (V + BW_ - 1) // BW_
    HD = D // 2

    def rne(y):
        b = lax.bitcast_convert_type(y, jnp.int32)
        return b + 0x7FFF + (lax.shift_right_logical(b, 16) & 1)

    def body(t_ref, out_ref):
        x = t_ref[...]
        for g in range(QB // 8):
            z_lo = jnp.concatenate(
                [x[0:HD, (8 * g + i) * CW:(8 * g + i + 1) * CW]
                 for i in range(8)], axis=0)
            z_hi = jnp.concatenate(
                [x[HD:D, (8 * g + i) * CW:(8 * g + i + 1) * CW]
                 for i in range(8)], axis=0)
            wl = rne(jnp.transpose(z_lo))
            wh = rne(jnp.transpose(z_hi))
            word = lax.shift_right_logical(wl, 16) | (wh & jnp.int32(-65536))
            out_ref[pl.ds(g * CW, CW), :] = word

    wide = pl.pallas_call(
        body,
        grid=(NBLK,),
        in_specs=[pl.BlockSpec((D, BW_), lambda m: (0, m))],
        out_specs=pl.BlockSpec((QB // 8 * CW, CW), lambda m: (m, 0)),
        out_shape=jax.ShapeDtypeStruct((NBLK * QB // 8 * CW, CW), jnp.int32),
    )(table.T)
    return wide.reshape(NBLK * BW_, HD)


def kernel(feats, weights, table):
    B, H = feats.shape
    V, D = table.shape
    CB = 2
    RB = B // _NW
    NCHUNK = RB // CB
    NBUF = 4
    RPC = CB * H
    HD = D // 2

    HP = 64
    fj = feats.astype(jnp.int32)
    fr = ((fj >> 10) << 10) + ((fj & 127) << 3) + ((fj >> 7) & 7)
    feats2 = fr.reshape(B // CB, RPC)
    weights2 = jnp.pad(weights, ((0, 0), (0, HP - H))).reshape(B // CB, CB * HP)
    table_pk = _relayout_table(table)

    mesh = plsc.VectorSubcoreMesh(core_axis_name="c", subcore_axis_name="s")

    @functools.partial(
        pl.kernel,
        out_type=jax.ShapeDtypeStruct((B, D), jnp.float32),
        mesh=mesh,
        scratch_types=[
            pltpu.VMEM((NCHUNK, RPC), jnp.int32),
            pltpu.VMEM((NCHUNK, CB * HP), jnp.float32),
            pltpu.VMEM((NBUF, RPC, HD), jnp.int32),
            pltpu.VMEM((RB, D), jnp.float32),
            pltpu.SemaphoreType.DMA,
        ],
        compiler_params=pltpu.CompilerParams(
            use_tc_tiling_on_sc=False, needs_layout_passes=False),
    )
    def run(feats_hbm, w_hbm, table_hbm, out_hbm, idx_v, w_v, rows_v, out_v, sem):
        wid = lax.axis_index("s") * _NC + lax.axis_index("c")
        chunk0 = wid * NCHUNK

        pltpu.sync_copy(feats_hbm.at[pl.ds(chunk0, NCHUNK)], idx_v)
        pltpu.sync_copy(w_hbm.at[pl.ds(chunk0, NCHUNK)], w_v)

        def fire(g, b):
            pltpu.async_copy(table_hbm.at[idx_v.at[g]], rows_v.at[b], sem)

        def wait(g, b):
            pltpu.make_async_copy(
                table_hbm.at[idx_v.at[g]], rows_v.at[b], sem).wait()

        def compute(g, b):
            for cb in range(CB):
                wvecs = [w_v[g, pl.ds(cb * HP + o, _LANES)] for o in (0, 16, 32, 40)]
                acc0 = jnp.zeros((_LANES,), jnp.float32)
                acc1 = jnp.zeros((_LANES,), jnp.float32)
                for l in range(H):
                    r = cb * H + l
                    w = wvecs[l // 16][l % 16] if l < 48 else wvecs[3][l - 40]
                    v = rows_v[b, r, :]
                    lo = plsc.bitcast(v << 16, jnp.float32)
                    hi = plsc.bitcast(v & jnp.int32(-65536), jnp.float32)
                    acc0 = acc0 + w * lo
                    acc1 = acc1 + w * hi
                out_v[g * CB + cb, pl.ds(0, _LANES)] = acc0
                out_v[g * CB + cb, pl.ds(_LANES, _LANES)] = acc1

        for b in range(NBUF):
            fire(b, b)

        @pl.loop(0, NCHUNK - NBUF, step=NBUF)
        def _(g0):
            for b in range(NBUF):
                g = g0 + b
                wait(g, b)
                compute(g, b)
                fire(g + NBUF, b)

        for b in range(NBUF):
            g = NCHUNK - NBUF + b
            wait(g, b)
            compute(g, b)

        pltpu.sync_copy(out_v, out_hbm.at[pl.ds(wid * RB, RB)])

    return run(feats2, weights2, table_pk)
